# baseline (speedup 1.0000x reference)
import functools

import jax
import jax.numpy as jnp
from jax import lax
from jax.experimental import pallas as pl
from jax.experimental.pallas import tpu as pltpu
from jax.experimental.pallas import tpu_sc as plsc

VOCAB = 1000000
EMBED_DIM = 32
BATCH = 16384
HIST_LEN = 50

NUM_WORKERS = 32
B_PER_W = BATCH // NUM_WORKERS
BT = BATCH // 128
PACK_B = 1024


def _sc_gather(idx_t, table):
    mesh = plsc.VectorSubcoreMesh(core_axis_name="c", subcore_axis_name="s")

    @functools.partial(
        pl.kernel,
        mesh=mesh,
        out_type=jax.ShapeDtypeStruct((HIST_LEN, BATCH, EMBED_DIM),
                                      jnp.float32),
        scratch_types=[
            pltpu.VMEM((HIST_LEN, B_PER_W), jnp.int32),
            pltpu.VMEM((B_PER_W, EMBED_DIM), jnp.float32),
            pltpu.VMEM((B_PER_W, EMBED_DIM), jnp.float32),
            pltpu.SemaphoreType.DMA,
            pltpu.SemaphoreType.DMA,
        ],
        compiler_params=pltpu.CompilerParams(use_tc_tiling_on_sc=False),
    )
    def k(idx_hbm, table_hbm, y_hbm, idx_v, g0, g1, gs0, gs1):
        wid = lax.axis_index("s") * 2 + lax.axis_index("c")
        b0 = wid * B_PER_W
        pltpu.sync_copy(idx_hbm.at[:, pl.ds(b0, B_PER_W)], idx_v)

        gbuf = (g0, g1)
        gsem = (gs0, gs1)

        def fire(h, p):
            pltpu.async_copy(table_hbm.at[idx_v.at[h]], gbuf[p], gsem[p])

        fire(0, 0)
        fire(1, 1)

        def body(i, carry):
            for p in (0, 1):
                h = 2 * i + p
                pltpu.make_async_copy(
                    table_hbm.at[pl.ds(0, B_PER_W)], gbuf[p], gsem[p]).wait()
                pltpu.sync_copy(gbuf[p], y_hbm.at[h, pl.ds(b0, B_PER_W)])
                @pl.when(h + 2 < HIST_LEN)
                def _():
                    fire(h + 2, p)
            return carry

        lax.fori_loop(0, HIST_LEN // 2, body, 0)

    return k(idx_t, table)


def _tc_pack_body(y_ref, x_ref):
    blk = y_ref[0]
    t = jnp.transpose(blk, (1, 0))
    t = t.reshape(EMBED_DIM // 8, 8, PACK_B // 128, 128)
    x_ref[0] = jnp.transpose(t, (0, 2, 1, 3))


def _tc_pack(y):
    grid = (HIST_LEN, BATCH // PACK_B)
    return pl.pallas_call(
        _tc_pack_body,
        grid=grid,
        in_specs=[pl.BlockSpec((1, PACK_B, EMBED_DIM),
                               lambda h, j: (h, j, 0))],
        out_specs=pl.BlockSpec(
            (1, EMBED_DIM // 8, PACK_B // 128, 8, 128),
            lambda h, j: (h, 0, j, 0, 0)),
        out_shape=jax.ShapeDtypeStruct(
            (HIST_LEN, EMBED_DIM // 8, BT, 8, 128), jnp.float32),
        compiler_params=pltpu.CompilerParams(
            dimension_semantics=("arbitrary", "arbitrary")),
    )(y)


def kernel(indices, embeddings):
    y = _sc_gather(indices.T, embeddings)
    x = _tc_pack(y)
    return x.transpose(2, 4, 0, 1, 3).reshape(BATCH, HIST_LEN, EMBED_DIM)


# speedup vs baseline: 1.0031x; 1.0031x over previous
import functools

import jax
import jax.numpy as jnp
from jax import lax
from jax.experimental import pallas as pl
from jax.experimental.pallas import tpu as pltpu
from jax.experimental.pallas import tpu_sc as plsc

VOCAB = 1000000
EMBED_DIM = 32
BATCH = 16384
HIST_LEN = 50

NUM_WORKERS = 32
B_PER_W = BATCH // NUM_WORKERS
BT = BATCH // 128
PACK_B = 1024
PACK_J = BATCH // PACK_B
PACK_STEPS = HIST_LEN * PACK_J


def _sc_gather(idx_t, table):
    mesh = plsc.VectorSubcoreMesh(core_axis_name="c", subcore_axis_name="s")

    @functools.partial(
        pl.kernel,
        mesh=mesh,
        out_type=jax.ShapeDtypeStruct((HIST_LEN, BATCH, EMBED_DIM),
                                      jnp.float32),
        scratch_types=[
            pltpu.VMEM((HIST_LEN, B_PER_W), jnp.int32),
            pltpu.VMEM((B_PER_W, EMBED_DIM), jnp.float32),
            pltpu.VMEM((B_PER_W, EMBED_DIM), jnp.float32),
            pltpu.SemaphoreType.DMA,
            pltpu.SemaphoreType.DMA,
        ],
        compiler_params=pltpu.CompilerParams(use_tc_tiling_on_sc=False),
    )
    def k(idx_hbm, table_hbm, y_hbm, idx_v, g0, g1, gs0, gs1):
        wid = lax.axis_index("s") * 2 + lax.axis_index("c")
        b0 = wid * B_PER_W
        pltpu.sync_copy(idx_hbm.at[:, pl.ds(b0, B_PER_W)], idx_v)

        gbuf = (g0, g1)
        gsem = (gs0, gs1)

        def fire(h, p):
            pltpu.async_copy(table_hbm.at[idx_v.at[h]], gbuf[p], gsem[p])

        fire(0, 0)
        fire(1, 1)

        def body(i, carry):
            for p in (0, 1):
                h = 2 * i + p
                pltpu.make_async_copy(
                    table_hbm.at[pl.ds(0, B_PER_W)], gbuf[p], gsem[p]).wait()
                pltpu.sync_copy(gbuf[p], y_hbm.at[h, pl.ds(b0, B_PER_W)])
                @pl.when(h + 2 < HIST_LEN)
                def _():
                    fire(h + 2, p)
            return carry

        lax.fori_loop(0, HIST_LEN // 2, body, 0)

    return k(idx_t, table)


def _tc_pack_body(y_hbm, x_ref, vb, sem):
    h = pl.program_id(0)
    j = pl.program_id(1)
    step = h * PACK_J + j
    p = step % 2

    def block_copy(hh, jj, q):
        return pltpu.make_async_copy(
            y_hbm.at[pl.ds(hh, 1), pl.ds(jj * PACK_B, PACK_B), :],
            vb.at[q], sem.at[q])

    @pl.when(step == 0)
    def _():
        block_copy(0, 0, 0).start()

    nstep = step + 1
    nh = nstep // PACK_J
    nj = nstep % PACK_J

    @pl.when(nstep < PACK_STEPS)
    def _():
        block_copy(nh, nj, 1 - p).start()

    block_copy(h, j, p).wait()
    t = jnp.transpose(vb[p, 0], (1, 0))
    for eg in range(EMBED_DIM // 8):
        for btl in range(PACK_B // 128):
            x_ref[0, eg, btl] = t[eg * 8:(eg + 1) * 8,
                                  btl * 128:(btl + 1) * 128]


def _tc_pack(y):
    return pl.pallas_call(
        _tc_pack_body,
        grid=(HIST_LEN, PACK_J),
        in_specs=[pl.BlockSpec(memory_space=pl.ANY)],
        out_specs=pl.BlockSpec(
            (1, EMBED_DIM // 8, PACK_B // 128, 8, 128),
            lambda h, j: (h, 0, j, 0, 0)),
        out_shape=jax.ShapeDtypeStruct(
            (HIST_LEN, EMBED_DIM // 8, BT, 8, 128), jnp.float32),
        scratch_shapes=[
            pltpu.VMEM((2, 1, PACK_B, EMBED_DIM), jnp.float32),
            pltpu.SemaphoreType.DMA((2,)),
        ],
        compiler_params=pltpu.CompilerParams(
            dimension_semantics=("arbitrary", "arbitrary")),
    )(y)


def kernel(indices, embeddings):
    y = _sc_gather(indices.T, embeddings)
    x = _tc_pack(y)
    return x.transpose(2, 4, 0, 1, 3).reshape(BATCH, HIST_LEN, EMBED_DIM)


# speedup vs baseline: 1.1389x; 1.1354x over previous
import functools

import jax
import jax.numpy as jnp
from jax import lax
from jax.experimental import pallas as pl
from jax.experimental.pallas import tpu as pltpu
from jax.experimental.pallas import tpu_sc as plsc

VOCAB = 1000000
EMBED_DIM = 32
BATCH = 16384
HIST_LEN = 50

NUM_WORKERS = 32
B_PER_W = BATCH // NUM_WORKERS
BT_PER_W = B_PER_W // 128


def _gather_call(idx_t, table):
    mesh = plsc.VectorSubcoreMesh(core_axis_name="c", subcore_axis_name="s")

    @functools.partial(
        pl.kernel,
        mesh=mesh,
        out_type=jax.ShapeDtypeStruct(
            (HIST_LEN, EMBED_DIM // 8, BATCH // 128, 8, 128), jnp.float32),
        scratch_types=[
            pltpu.VMEM((HIST_LEN, B_PER_W), jnp.int32),
            pltpu.VMEM((B_PER_W, EMBED_DIM), jnp.float32),
            pltpu.VMEM((B_PER_W, EMBED_DIM), jnp.float32),
            pltpu.VMEM((EMBED_DIM // 8, BT_PER_W, 8, 128), jnp.float32),
            pltpu.VMEM((EMBED_DIM // 8, BT_PER_W, 8, 128), jnp.float32),
            pltpu.SemaphoreType.DMA,
            pltpu.SemaphoreType.DMA,
            pltpu.SemaphoreType.DMA,
            pltpu.SemaphoreType.DMA,
        ],
        compiler_params=pltpu.CompilerParams(
            use_tc_tiling_on_sc=False, needs_layout_passes=False),
    )
    def k(idx_hbm, table_hbm, out_hbm, idx_v, g0, g1, t0, t1,
          gs0, gs1, ss0, ss1):
        wid = lax.axis_index("s") * 2 + lax.axis_index("c")
        b0 = wid * B_PER_W
        pltpu.sync_copy(idx_hbm.at[:, pl.ds(b0, B_PER_W)], idx_v)

        gbuf = (g0, g1)
        tbuf = (t0, t1)
        gsem = (gs0, gs1)
        ssem = (ss0, ss1)
        lane = lax.iota(jnp.int32, 16)
        rowvs = [(bt * 128 + blg * 16) + lane
                 for bt in range(BT_PER_W) for blg in range(8)]
        colvs = [jnp.full((16,), e, jnp.int32) for e in range(EMBED_DIM)]

        def fire_gather(h, p):
            pltpu.async_copy(table_hbm.at[idx_v.at[h]], gbuf[p], gsem[p])

        def out_slice(h):
            return out_hbm.at[h, :, pl.ds(wid * BT_PER_W, BT_PER_W), :, :]

        fire_gather(0, 0)
        fire_gather(1, 1)

        def body(i, carry):
            for p in (0, 1):
                h = 2 * i + p
                pltpu.make_async_copy(
                    table_hbm.at[pl.ds(0, B_PER_W)], gbuf[p], gsem[p]).wait()
                @pl.when(i > 0)
                def _():
                    pltpu.make_async_copy(
                        tbuf[p], out_slice(h), ssem[p]).wait()
                g, t = gbuf[p], tbuf[p]
                for eg in range(EMBED_DIM // 8):
                    for es in range(8):
                        colv = colvs[eg * 8 + es]
                        for bt in range(BT_PER_W):
                            for blg in range(8):
                                t[eg, bt, es, pl.ds(blg * 16, 16)] = (
                                    plsc.load_gather(
                                        g, [rowvs[bt * 8 + blg], colv]))
                @pl.when(h + 2 < HIST_LEN)
                def _():
                    fire_gather(h + 2, p)
                pltpu.async_copy(tbuf[p], out_slice(h), ssem[p])
            return carry

        lax.fori_loop(0, HIST_LEN // 2, body, 0)
        for p in (0, 1):
            pltpu.make_async_copy(
                tbuf[p], out_slice(HIST_LEN - 2 + p), ssem[p]).wait()

    return k(idx_t, table)


def kernel(indices, embeddings):
    x = _gather_call(indices.T, embeddings)
    return x.transpose(2, 4, 0, 1, 3).reshape(BATCH, HIST_LEN, EMBED_DIM)
